# baseline (speedup 1.0000x reference)
import jax
import jax.numpy as jnp
from jax import lax
from jax.experimental import pallas as pl
from jax.experimental.pallas import tpu as pltpu
from jax.experimental.pallas import tpu_sc as plsc

BATCH = 16384
EMB = 128
HID = EMB // 2
NWORKER = 32
PER_W = BATCH // NWORKER
WIN = 128
NWIN = PER_W // WIN
BLK = 16384
ROWTILES = BLK // 128


def _sc_gather(user, item, user_table, item_table):
    mesh = plsc.VectorSubcoreMesh(core_axis_name="core", subcore_axis_name="subcore")
    out_type = (
        jax.ShapeDtypeStruct((BATCH, EMB), jnp.float32),
        jax.ShapeDtypeStruct((BATCH, EMB), jnp.float32),
    )
    scratch = [
        pltpu.VMEM((PER_W,), jnp.int32),
        pltpu.VMEM((PER_W,), jnp.int32),
        pltpu.VMEM((3, WIN, EMB), jnp.float32),
        pltpu.VMEM((3, WIN, EMB), jnp.float32),
    ] + [pltpu.SemaphoreType.DMA] * 12

    @pl.kernel(out_type=out_type, mesh=mesh, scratch_types=scratch)
    def gather_kernel(u_hbm, i_hbm, ut_hbm, it_hbm, uo_hbm, io_hbm,
                      uidx, iidx, ubuf, ibuf,
                      gu0, gu1, gu2, gi0, gi1, gi2,
                      wu0, wu1, wu2, wi0, wi1, wi2):
        gu = (gu0, gu1, gu2)
        gi = (gi0, gi1, gi2)
        wu = (wu0, wu1, wu2)
        wi = (wi0, wi1, wi2)
        wid = lax.axis_index("subcore") * 2 + lax.axis_index("core")
        base = wid * PER_W

        pltpu.sync_copy(u_hbm.at[pl.ds(base, PER_W)], uidx)
        pltpu.sync_copy(i_hbm.at[pl.ds(base, PER_W)], iidx)

        def start_gather(w):
            b = w % 3
            cu = pltpu.async_copy(
                ut_hbm.at[uidx.at[pl.ds(w * WIN, WIN)]], ubuf.at[b], gu[b])
            ci = pltpu.async_copy(
                it_hbm.at[iidx.at[pl.ds(w * WIN, WIN)]], ibuf.at[b], gi[b])
            return cu, ci

        def start_write(w):
            b = w % 3
            cu = pltpu.async_copy(
                ubuf.at[b], uo_hbm.at[pl.ds(base + w * WIN, WIN)], wu[b])
            ci = pltpu.async_copy(
                ibuf.at[b], io_hbm.at[pl.ds(base + w * WIN, WIN)], wi[b])
            return cu, ci

        gathers = [None] * NWIN
        writes = [None] * NWIN
        gathers[0] = start_gather(0)
        gathers[1] = start_gather(1)
        for w in range(NWIN):
            if w + 2 < NWIN:
                if w - 1 >= 0:
                    writes[w - 1][0].wait()
                    writes[w - 1][1].wait()
                gathers[w + 2] = start_gather(w + 2)
            gathers[w][0].wait()
            gathers[w][1].wait()
            writes[w] = start_write(w)
        for w in range(max(0, NWIN - 3), NWIN):
            writes[w][0].wait()
            writes[w][1].wait()

    return gather_kernel(user, item, user_table, item_table)


def _dot_t(x, w):
    return lax.dot_general(x, w, (((1,), (1,)), ((), ())),
                           preferred_element_type=jnp.float32)


def _mlp_body(ue_ref, ie_ref, w1_ref, b1_ref, w2_ref, b2_ref,
              w3_ref, b3_ref, o_ref):
    h = _dot_t(ue_ref[...], w1_ref[:, :EMB])
    h = h + _dot_t(ie_ref[...], w1_ref[:, EMB:])
    h = jnp.maximum(h + b1_ref[...], 0.0)
    h2 = jnp.maximum(_dot_t(h, w2_ref[...]) + b2_ref[...], 0.0)
    res = jnp.sum(h2 * w3_ref[...], axis=1) + b3_ref[0, 0]
    o_ref[...] = res.reshape(ROWTILES, 128)


def _tc_mlp(ue, ie, W1, b1, W2, b2, w3, b3):
    grid = (BATCH // BLK,)
    return pl.pallas_call(
        _mlp_body,
        grid=grid,
        in_specs=[
            pl.BlockSpec((BLK, EMB), lambda i: (i, 0)),
            pl.BlockSpec((BLK, EMB), lambda i: (i, 0)),
            pl.BlockSpec((EMB, 2 * EMB), lambda i: (0, 0)),
            pl.BlockSpec((1, EMB), lambda i: (0, 0)),
            pl.BlockSpec((HID, EMB), lambda i: (0, 0)),
            pl.BlockSpec((1, HID), lambda i: (0, 0)),
            pl.BlockSpec((1, HID), lambda i: (0, 0)),
            pl.BlockSpec((1, 1), lambda i: (0, 0)),
        ],
        out_specs=pl.BlockSpec((ROWTILES, 128), lambda i: (i, 0)),
        out_shape=jax.ShapeDtypeStruct((BATCH // 128, 128), jnp.float32),
    )(ue, ie, W1, b1, W2, b2, w3, b3)


def kernel(user, item, user_table, item_table, W1, b1, W2, b2, W3, b3):
    ue, ie = _sc_gather(user.astype(jnp.int32), item.astype(jnp.int32),
                        user_table, item_table)
    out = _tc_mlp(
        ue, ie,
        W1, b1.reshape(1, EMB),
        W2, b2.reshape(1, HID),
        W3.reshape(1, HID), b3.reshape(1, 1),
    )
    return out.reshape(BATCH)


# speedup vs baseline: 1.0459x; 1.0459x over previous
import jax
import jax.numpy as jnp
from jax import lax
from jax.experimental import pallas as pl
from jax.experimental.pallas import tpu as pltpu
from jax.experimental.pallas import tpu_sc as plsc

BATCH = 16384
EMB = 128
HID = EMB // 2
NWORKER = 32
PER_W = BATCH // NWORKER
WIN = 128
NWIN = PER_W // WIN
BLK = 8192
ROWTILES = BLK // 128


def _sc_gather(user, item, user_table, item_table):
    mesh = plsc.VectorSubcoreMesh(core_axis_name="core", subcore_axis_name="subcore")
    out_type = (
        jax.ShapeDtypeStruct((BATCH, EMB), jnp.float32),
        jax.ShapeDtypeStruct((BATCH, EMB), jnp.float32),
    )
    scratch = [
        pltpu.VMEM((PER_W,), jnp.int32),
        pltpu.VMEM((PER_W,), jnp.int32),
        pltpu.VMEM((3, WIN, EMB), jnp.float32),
        pltpu.VMEM((3, WIN, EMB), jnp.float32),
    ] + [pltpu.SemaphoreType.DMA] * 14

    @pl.kernel(out_type=out_type, mesh=mesh, scratch_types=scratch)
    def gather_kernel(u_hbm, i_hbm, ut_hbm, it_hbm, uo_hbm, io_hbm,
                      uidx, iidx, ubuf, ibuf,
                      gu0, gu1, gu2, gi0, gi1, gi2,
                      wu0, wu1, wu2, wi0, wi1, wi2, xu, xi):
        gu = (gu0, gu1, gu2)
        gi = (gi0, gi1, gi2)
        wu = (wu0, wu1, wu2)
        wi = (wi0, wi1, wi2)
        wid = lax.axis_index("subcore") * 2 + lax.axis_index("core")
        base = wid * PER_W

        lu = pltpu.async_copy(u_hbm.at[pl.ds(base, PER_W)], uidx, xu)
        li = pltpu.async_copy(i_hbm.at[pl.ds(base, PER_W)], iidx, xi)
        lu.wait()
        li.wait()

        def start_gather(w):
            b = w % 3
            cu = pltpu.async_copy(
                ut_hbm.at[uidx.at[pl.ds(w * WIN, WIN)]], ubuf.at[b], gu[b])
            ci = pltpu.async_copy(
                it_hbm.at[iidx.at[pl.ds(w * WIN, WIN)]], ibuf.at[b], gi[b])
            return cu, ci

        def start_write(w):
            b = w % 3
            cu = pltpu.async_copy(
                ubuf.at[b], uo_hbm.at[pl.ds(base + w * WIN, WIN)], wu[b])
            ci = pltpu.async_copy(
                ibuf.at[b], io_hbm.at[pl.ds(base + w * WIN, WIN)], wi[b])
            return cu, ci

        gathers = [None] * NWIN
        writes = [None] * NWIN
        gathers[0] = start_gather(0)
        gathers[1] = start_gather(1)
        for w in range(NWIN):
            if w + 2 < NWIN:
                if w - 1 >= 0:
                    writes[w - 1][0].wait()
                    writes[w - 1][1].wait()
                gathers[w + 2] = start_gather(w + 2)
            gathers[w][0].wait()
            gathers[w][1].wait()
            writes[w] = start_write(w)
        for w in range(max(0, NWIN - 3), NWIN):
            writes[w][0].wait()
            writes[w][1].wait()

    return gather_kernel(user, item, user_table, item_table)


def _dot_t(x, w):
    return lax.dot_general(x, w, (((1,), (1,)), ((), ())),
                           preferred_element_type=jnp.float32)


def _mlp_body(ue_ref, ie_ref, w1_ref, b1_ref, w2_ref, b2_ref,
              w3_ref, b3_ref, o_ref):
    h = _dot_t(ue_ref[...], w1_ref[:, :EMB])
    h = h + _dot_t(ie_ref[...], w1_ref[:, EMB:])
    h = jnp.maximum(h + b1_ref[...], 0.0)
    h2 = jnp.maximum(_dot_t(h, w2_ref[...]) + b2_ref[...], 0.0)
    res = jnp.sum(h2 * w3_ref[...], axis=1) + b3_ref[0, 0]
    o_ref[...] = res.reshape(ROWTILES, 128)


def _tc_mlp(ue, ie, W1, b1, W2, b2, w3, b3):
    grid = (BATCH // BLK,)
    return pl.pallas_call(
        _mlp_body,
        grid=grid,
        in_specs=[
            pl.BlockSpec((BLK, EMB), lambda i: (i, 0)),
            pl.BlockSpec((BLK, EMB), lambda i: (i, 0)),
            pl.BlockSpec((EMB, 2 * EMB), lambda i: (0, 0)),
            pl.BlockSpec((1, EMB), lambda i: (0, 0)),
            pl.BlockSpec((HID, EMB), lambda i: (0, 0)),
            pl.BlockSpec((1, HID), lambda i: (0, 0)),
            pl.BlockSpec((1, HID), lambda i: (0, 0)),
            pl.BlockSpec((1, 1), lambda i: (0, 0)),
        ],
        out_specs=pl.BlockSpec((ROWTILES, 128), lambda i: (i, 0)),
        out_shape=jax.ShapeDtypeStruct((BATCH // 128, 128), jnp.float32),
    )(ue, ie, W1, b1, W2, b2, w3, b3)


def kernel(user, item, user_table, item_table, W1, b1, W2, b2, W3, b3):
    ue, ie = _sc_gather(user.astype(jnp.int32), item.astype(jnp.int32),
                        user_table, item_table)
    out = _tc_mlp(
        ue, ie,
        W1, b1.reshape(1, EMB),
        W2, b2.reshape(1, HID),
        W3.reshape(1, HID), b3.reshape(1, 1),
    )
    return out.reshape(BATCH)
